# initial kernel scaffold (unmeasured)
import jax
import jax.numpy as jnp
from jax import lax
from jax.experimental import pallas as pl
from jax.experimental.pallas import tpu as pltpu

M = 2048
D = 2048
HALF = M // 2


def kernel(partial, resid, gamma):
    partial = partial.reshape(M, D)
    gamma = gamma.reshape(1, D)

    def body(p_ref, r_ref, g_ref, out_ref,
             xsend, xrecv, ysend, yrecv,
             sem_sx, sem_rx, sem_sy, sem_ry):
        my_x = lax.axis_index("x")
        my_y = lax.axis_index("y")
        x_nbr = (1 - my_x, my_y)
        y_nbr = (my_x, 1 - my_y)
        row0 = my_y * HALF
        other_row0 = (1 - my_y) * HALF

        barrier_sem = pltpu.get_barrier_semaphore()
        for nbr in (x_nbr, y_nbr):
            pl.semaphore_signal(
                barrier_sem, inc=1,
                device_id=nbr, device_id_type=pl.DeviceIdType.MESH,
            )
        pl.semaphore_wait(barrier_sem, 2)

        xsend[...] = p_ref[pl.ds(row0, HALF), :].astype(jnp.bfloat16)
        rdma_x = pltpu.make_async_remote_copy(
            src_ref=xsend, dst_ref=xrecv,
            send_sem=sem_sx, recv_sem=sem_rx,
            device_id=x_nbr, device_id_type=pl.DeviceIdType.MESH,
        )
        rdma_x.start()
        rdma_x.wait()

        y = (p_ref[pl.ds(row0, HALF), :]
             + xrecv[...].astype(jnp.float32)
             + r_ref[pl.ds(row0, HALF), :])
        inv = lax.rsqrt(jnp.mean(y * y, axis=-1, keepdims=True) + 1e-6)
        out_half = y * inv * g_ref[...]
        out_ref[pl.ds(row0, HALF), :] = out_half
        ysend[...] = out_half.astype(jnp.bfloat16)

        rdma_y = pltpu.make_async_remote_copy(
            src_ref=ysend, dst_ref=yrecv,
            send_sem=sem_sy, recv_sem=sem_ry,
            device_id=y_nbr, device_id_type=pl.DeviceIdType.MESH,
        )
        rdma_y.start()
        rdma_y.wait()
        out_ref[pl.ds(other_row0, HALF), :] = yrecv[...].astype(jnp.float32)

    return pl.pallas_call(
        body,
        out_shape=jax.ShapeDtypeStruct((M, D), jnp.float32),
        in_specs=[
            pl.BlockSpec(memory_space=pltpu.VMEM),
            pl.BlockSpec(memory_space=pltpu.VMEM),
            pl.BlockSpec(memory_space=pltpu.VMEM),
        ],
        out_specs=pl.BlockSpec(memory_space=pltpu.VMEM),
        scratch_shapes=[
            pltpu.VMEM((HALF, D), jnp.bfloat16),
            pltpu.VMEM((HALF, D), jnp.bfloat16),
            pltpu.VMEM((HALF, D), jnp.bfloat16),
            pltpu.VMEM((HALF, D), jnp.bfloat16),
            pltpu.SemaphoreType.DMA,
            pltpu.SemaphoreType.DMA,
            pltpu.SemaphoreType.DMA,
            pltpu.SemaphoreType.DMA,
        ],
        compiler_params=pltpu.CompilerParams(collective_id=0),
    )(partial, resid, gamma)


# baseline (device time: 120028 ns/iter reference)
import jax
import jax.numpy as jnp
from jax import lax
from jax.experimental import pallas as pl
from jax.experimental.pallas import tpu as pltpu

M = 2048
D = 2048
HALF = M // 2


def kernel(partial, resid, gamma):
    partial = partial.reshape(M, D)
    gamma = gamma.reshape(1, D)

    def body(p_ref, r_ref, g_ref, out_ref,
             p_half, r_half, xsend, xrecv, ysend, yrecv,
             sem_local, sem_sx, sem_rx, sem_sy, sem_ry):
        my_x = lax.axis_index("x")
        my_y = lax.axis_index("y")
        x_nbr = (1 - my_x, my_y)
        y_nbr = (my_x, 1 - my_y)
        row0 = my_y * HALF
        other_row0 = (1 - my_y) * HALF

        cp_p = pltpu.make_async_copy(
            p_ref.at[pl.ds(row0, HALF), :], p_half, sem_local)
        cp_p.start()
        cp_r = pltpu.make_async_copy(
            r_ref.at[pl.ds(row0, HALF), :], r_half, sem_local)
        cp_p.wait()
        cp_r.start()

        barrier_sem = pltpu.get_barrier_semaphore()
        for nbr in (x_nbr, y_nbr):
            pl.semaphore_signal(
                barrier_sem, inc=1,
                device_id=nbr, device_id_type=pl.DeviceIdType.MESH,
            )
        pl.semaphore_wait(barrier_sem, 2)

        xsend[...] = p_half[...].astype(jnp.bfloat16)
        rdma_x = pltpu.make_async_remote_copy(
            src_ref=xsend, dst_ref=xrecv,
            send_sem=sem_sx, recv_sem=sem_rx,
            device_id=x_nbr, device_id_type=pl.DeviceIdType.MESH,
        )
        rdma_x.start()
        rdma_x.wait()
        cp_r.wait()

        y = p_half[...] + xrecv[...].astype(jnp.float32) + r_half[...]
        inv = lax.rsqrt(jnp.mean(y * y, axis=-1, keepdims=True) + 1e-6)
        p_half[...] = y * inv * g_ref[...]
        cp_out0 = pltpu.make_async_copy(
            p_half, out_ref.at[pl.ds(row0, HALF), :], sem_local)
        cp_out0.start()
        ysend[...] = p_half[...].astype(jnp.bfloat16)

        rdma_y = pltpu.make_async_remote_copy(
            src_ref=ysend, dst_ref=yrecv,
            send_sem=sem_sy, recv_sem=sem_ry,
            device_id=y_nbr, device_id_type=pl.DeviceIdType.MESH,
        )
        rdma_y.start()
        rdma_y.wait()
        r_half[...] = yrecv[...].astype(jnp.float32)
        cp_out0.wait()
        cp_out1 = pltpu.make_async_copy(
            r_half, out_ref.at[pl.ds(other_row0, HALF), :], sem_local)
        cp_out1.start()
        cp_out1.wait()

    return pl.pallas_call(
        body,
        out_shape=jax.ShapeDtypeStruct((M, D), jnp.float32),
        in_specs=[
            pl.BlockSpec(memory_space=pl.ANY),
            pl.BlockSpec(memory_space=pl.ANY),
            pl.BlockSpec(memory_space=pltpu.VMEM),
        ],
        out_specs=pl.BlockSpec(memory_space=pl.ANY),
        scratch_shapes=[
            pltpu.VMEM((HALF, D), jnp.float32),
            pltpu.VMEM((HALF, D), jnp.float32),
            pltpu.VMEM((HALF, D), jnp.bfloat16),
            pltpu.VMEM((HALF, D), jnp.bfloat16),
            pltpu.VMEM((HALF, D), jnp.bfloat16),
            pltpu.VMEM((HALF, D), jnp.bfloat16),
            pltpu.SemaphoreType.DMA,
            pltpu.SemaphoreType.DMA,
            pltpu.SemaphoreType.DMA,
            pltpu.SemaphoreType.DMA,
            pltpu.SemaphoreType.DMA,
        ],
        compiler_params=pltpu.CompilerParams(
            collective_id=0,
            vmem_limit_bytes=60 * 1024 * 1024,
        ),
    )(partial, resid, gamma)


# device time: 72855 ns/iter; 1.6475x vs baseline; 1.6475x over previous
import jax
import jax.numpy as jnp
from jax import lax
from jax.experimental import pallas as pl
from jax.experimental.pallas import tpu as pltpu

M = 2048
D = 2048
HALF = M // 2
NCHUNK = 8
ROWS = HALF // NCHUNK


def kernel(partial, resid, gamma):
    partial = partial.reshape(M, D)
    gamma = gamma.reshape(1, D)

    def body(p_ref, r_ref, g_ref, out_ref,
             p_half, r_half, xsend, xrecv, ysend, yrecv,
             sem_lp, sem_lr, sem_sx, sem_rx, sem_sy, sem_ry):
        my_x = lax.axis_index("x")
        my_y = lax.axis_index("y")
        x_nbr = (1 - my_x, my_y)
        y_nbr = (my_x, 1 - my_y)
        row0 = my_y * HALF
        other_row0 = (1 - my_y) * HALF

        def chunk(buf, c):
            return buf.at[pl.ds(c * ROWS, ROWS), :]

        cp_p = []
        cp_r = []
        for c in range(NCHUNK):
            cp = pltpu.make_async_copy(
                p_ref.at[pl.ds(row0 + c * ROWS, ROWS), :],
                chunk(p_half, c), sem_lp.at[c])
            cp.start()
            cp_p.append(cp)
            cr = pltpu.make_async_copy(
                r_ref.at[pl.ds(row0 + c * ROWS, ROWS), :],
                chunk(r_half, c), sem_lr.at[c])
            cr.start()
            cp_r.append(cr)

        barrier_sem = pltpu.get_barrier_semaphore()
        for nbr in (x_nbr, y_nbr):
            pl.semaphore_signal(
                barrier_sem, inc=1,
                device_id=nbr, device_id_type=pl.DeviceIdType.MESH,
            )
        pl.semaphore_wait(barrier_sem, 2)

        rdma_x = []
        for c in range(NCHUNK):
            cp_p[c].wait()
            chunk(xsend, c)[...] = chunk(p_half, c)[...].astype(jnp.bfloat16)
            r = pltpu.make_async_remote_copy(
                src_ref=chunk(xsend, c), dst_ref=chunk(xrecv, c),
                send_sem=sem_sx.at[c], recv_sem=sem_rx.at[c],
                device_id=x_nbr, device_id_type=pl.DeviceIdType.MESH,
            )
            r.start()
            rdma_x.append(r)

        rdma_y = []
        cp_out0 = []
        for c in range(NCHUNK):
            rdma_x[c].wait()
            cp_r[c].wait()
            y = (chunk(p_half, c)[...]
                 + chunk(xrecv, c)[...].astype(jnp.float32)
                 + chunk(r_half, c)[...])
            inv = lax.rsqrt(jnp.mean(y * y, axis=-1, keepdims=True) + 1e-6)
            chunk(p_half, c)[...] = y * inv * g_ref[...]
            co = pltpu.make_async_copy(
                chunk(p_half, c),
                out_ref.at[pl.ds(row0 + c * ROWS, ROWS), :],
                sem_lp.at[c])
            co.start()
            cp_out0.append(co)
            chunk(ysend, c)[...] = chunk(p_half, c)[...].astype(jnp.bfloat16)
            r = pltpu.make_async_remote_copy(
                src_ref=chunk(ysend, c), dst_ref=chunk(yrecv, c),
                send_sem=sem_sy.at[c], recv_sem=sem_ry.at[c],
                device_id=y_nbr, device_id_type=pl.DeviceIdType.MESH,
            )
            r.start()
            rdma_y.append(r)

        cp_out1 = []
        for c in range(NCHUNK):
            rdma_y[c].wait()
            chunk(r_half, c)[...] = chunk(yrecv, c)[...].astype(jnp.float32)
            co = pltpu.make_async_copy(
                chunk(r_half, c),
                out_ref.at[pl.ds(other_row0 + c * ROWS, ROWS), :],
                sem_lr.at[c])
            co.start()
            cp_out1.append(co)

        for c in range(NCHUNK):
            cp_out0[c].wait()
            cp_out1[c].wait()

    return pl.pallas_call(
        body,
        out_shape=jax.ShapeDtypeStruct((M, D), jnp.float32),
        in_specs=[
            pl.BlockSpec(memory_space=pl.ANY),
            pl.BlockSpec(memory_space=pl.ANY),
            pl.BlockSpec(memory_space=pltpu.VMEM),
        ],
        out_specs=pl.BlockSpec(memory_space=pl.ANY),
        scratch_shapes=[
            pltpu.VMEM((HALF, D), jnp.float32),
            pltpu.VMEM((HALF, D), jnp.float32),
            pltpu.VMEM((HALF, D), jnp.bfloat16),
            pltpu.VMEM((HALF, D), jnp.bfloat16),
            pltpu.VMEM((HALF, D), jnp.bfloat16),
            pltpu.VMEM((HALF, D), jnp.bfloat16),
            pltpu.SemaphoreType.DMA((NCHUNK,)),
            pltpu.SemaphoreType.DMA((NCHUNK,)),
            pltpu.SemaphoreType.DMA((NCHUNK,)),
            pltpu.SemaphoreType.DMA((NCHUNK,)),
            pltpu.SemaphoreType.DMA((NCHUNK,)),
            pltpu.SemaphoreType.DMA((NCHUNK,)),
        ],
        compiler_params=pltpu.CompilerParams(
            collective_id=0,
            vmem_limit_bytes=60 * 1024 * 1024,
        ),
    )(partial, resid, gamma)


# device time: 69849 ns/iter; 1.7184x vs baseline; 1.0430x over previous
import jax
import jax.numpy as jnp
from jax import lax
from jax.experimental import pallas as pl
from jax.experimental.pallas import tpu as pltpu

M = 2048
D = 2048
HALF = M // 2
NCHUNK = 16
ROWS = HALF // NCHUNK


def kernel(partial, resid, gamma):
    partial = partial.reshape(M, D)
    gamma = gamma.reshape(1, D)

    def body(p_ref, r_ref, g_ref, out_ref,
             p_half, r_half, xsend, xrecv, ysend, yrecv,
             sem_lp, sem_lr, sem_sx, sem_rx, sem_sy, sem_ry):
        my_x = lax.axis_index("x")
        my_y = lax.axis_index("y")
        x_nbr = (1 - my_x, my_y)
        y_nbr = (my_x, 1 - my_y)
        row0 = my_y * HALF
        other_row0 = (1 - my_y) * HALF

        def chunk(buf, c):
            return buf.at[pl.ds(c * ROWS, ROWS), :]

        cp_p = []
        cp_r = []
        for c in range(NCHUNK):
            cp = pltpu.make_async_copy(
                p_ref.at[pl.ds(row0 + c * ROWS, ROWS), :],
                chunk(p_half, c), sem_lp.at[c])
            cp.start()
            cp_p.append(cp)
            cr = pltpu.make_async_copy(
                r_ref.at[pl.ds(row0 + c * ROWS, ROWS), :],
                chunk(r_half, c), sem_lr.at[c])
            cr.start()
            cp_r.append(cr)

        barrier_sem = pltpu.get_barrier_semaphore()
        for nbr in (x_nbr, y_nbr):
            pl.semaphore_signal(
                barrier_sem, inc=1,
                device_id=nbr, device_id_type=pl.DeviceIdType.MESH,
            )
        pl.semaphore_wait(barrier_sem, 2)

        rdma_x = []
        for c in range(NCHUNK):
            cp_p[c].wait()
            chunk(xsend, c)[...] = chunk(p_half, c)[...].astype(jnp.bfloat16)
            r = pltpu.make_async_remote_copy(
                src_ref=chunk(xsend, c), dst_ref=chunk(xrecv, c),
                send_sem=sem_sx.at[c], recv_sem=sem_rx.at[c],
                device_id=x_nbr, device_id_type=pl.DeviceIdType.MESH,
            )
            r.start()
            rdma_x.append(r)

        rdma_y = []
        cp_out0 = []
        for c in range(NCHUNK):
            rdma_x[c].wait()
            cp_r[c].wait()
            y = (chunk(p_half, c)[...]
                 + chunk(xrecv, c)[...].astype(jnp.float32)
                 + chunk(r_half, c)[...])
            inv = lax.rsqrt(jnp.mean(y * y, axis=-1, keepdims=True) + 1e-6)
            chunk(p_half, c)[...] = y * inv * g_ref[...]
            chunk(ysend, c)[...] = chunk(p_half, c)[...].astype(jnp.bfloat16)
            r = pltpu.make_async_remote_copy(
                src_ref=chunk(ysend, c), dst_ref=chunk(yrecv, c),
                send_sem=sem_sy.at[c], recv_sem=sem_ry.at[c],
                device_id=y_nbr, device_id_type=pl.DeviceIdType.MESH,
            )
            r.start()
            rdma_y.append(r)
            co = pltpu.make_async_copy(
                chunk(p_half, c),
                out_ref.at[pl.ds(row0 + c * ROWS, ROWS), :],
                sem_lp.at[c])
            co.start()
            cp_out0.append(co)

        cp_out1 = []
        for c in range(NCHUNK):
            rdma_y[c].wait()
            chunk(r_half, c)[...] = chunk(yrecv, c)[...].astype(jnp.float32)
            co = pltpu.make_async_copy(
                chunk(r_half, c),
                out_ref.at[pl.ds(other_row0 + c * ROWS, ROWS), :],
                sem_lr.at[c])
            co.start()
            cp_out1.append(co)

        for c in range(NCHUNK):
            cp_out0[c].wait()
            cp_out1[c].wait()

    return pl.pallas_call(
        body,
        out_shape=jax.ShapeDtypeStruct((M, D), jnp.float32),
        in_specs=[
            pl.BlockSpec(memory_space=pl.ANY),
            pl.BlockSpec(memory_space=pl.ANY),
            pl.BlockSpec(memory_space=pltpu.VMEM),
        ],
        out_specs=pl.BlockSpec(memory_space=pl.ANY),
        scratch_shapes=[
            pltpu.VMEM((HALF, D), jnp.float32),
            pltpu.VMEM((HALF, D), jnp.float32),
            pltpu.VMEM((HALF, D), jnp.bfloat16),
            pltpu.VMEM((HALF, D), jnp.bfloat16),
            pltpu.VMEM((HALF, D), jnp.bfloat16),
            pltpu.VMEM((HALF, D), jnp.bfloat16),
            pltpu.SemaphoreType.DMA((NCHUNK,)),
            pltpu.SemaphoreType.DMA((NCHUNK,)),
            pltpu.SemaphoreType.DMA((NCHUNK,)),
            pltpu.SemaphoreType.DMA((NCHUNK,)),
            pltpu.SemaphoreType.DMA((NCHUNK,)),
            pltpu.SemaphoreType.DMA((NCHUNK,)),
        ],
        compiler_params=pltpu.CompilerParams(
            collective_id=0,
            vmem_limit_bytes=60 * 1024 * 1024,
        ),
    )(partial, resid, gamma)


# device time: 69798 ns/iter; 1.7196x vs baseline; 1.0007x over previous
import jax
import jax.numpy as jnp
from jax import lax
from jax.experimental import pallas as pl
from jax.experimental.pallas import tpu as pltpu

M = 2048
D = 2048
HALF = M // 2
NCHUNK = 16
ROWS = HALF // NCHUNK


def kernel(partial, resid, gamma):
    partial = partial.reshape(M, D)
    gamma = gamma.reshape(1, D)

    def body(p_ref, r_ref, g_ref, out_ref,
             p_half, r_half, xsend, xrecv, ysend, yrecv,
             sem_lp, sem_lr, sem_sx, sem_rx, sem_sy, sem_ry):
        my_x = lax.axis_index("x")
        my_y = lax.axis_index("y")
        x_nbr = (1 - my_x, my_y)
        y_nbr = (my_x, 1 - my_y)
        row0 = my_y * HALF
        other_row0 = (1 - my_y) * HALF

        def chunk(buf, c):
            return buf.at[pl.ds(c * ROWS, ROWS), :]

        cp_p = []
        cp_r = []
        for c in range(NCHUNK):
            cp = pltpu.make_async_copy(
                p_ref.at[pl.ds(row0 + c * ROWS, ROWS), :],
                chunk(p_half, c), sem_lp.at[c])
            cp.start()
            cp_p.append(cp)
            cr = pltpu.make_async_copy(
                r_ref.at[pl.ds(row0 + c * ROWS, ROWS), :],
                chunk(r_half, c), sem_lr.at[c])
            cr.start()
            cp_r.append(cr)

        barrier_sem = pltpu.get_barrier_semaphore()
        for nbr in (x_nbr, y_nbr):
            pl.semaphore_signal(
                barrier_sem, inc=1,
                device_id=nbr, device_id_type=pl.DeviceIdType.MESH,
            )
        pl.semaphore_wait(barrier_sem, 2)

        rdma_x = []
        for c in range(NCHUNK):
            cp_p[c].wait()
            chunk(xsend, c)[...] = chunk(p_half, c)[...].astype(jnp.bfloat16)
            r = pltpu.make_async_remote_copy(
                src_ref=chunk(xsend, c), dst_ref=chunk(xrecv, c),
                send_sem=sem_sx.at[c], recv_sem=sem_rx.at[c],
                device_id=x_nbr, device_id_type=pl.DeviceIdType.MESH,
            )
            r.start()
            rdma_x.append(r)

        gb = g_ref[...].astype(jnp.bfloat16)
        rdma_y = []
        cp_out0 = []
        cp_out1 = []

        def drain_y(c):
            rdma_y[c].wait()
            chunk(r_half, c)[...] = chunk(yrecv, c)[...].astype(jnp.float32)
            co = pltpu.make_async_copy(
                chunk(r_half, c),
                out_ref.at[pl.ds(other_row0 + c * ROWS, ROWS), :],
                sem_lr.at[c])
            co.start()
            cp_out1.append(co)

        for c in range(NCHUNK):
            rdma_x[c].wait()
            cp_r[c].wait()
            y = (chunk(xsend, c)[...] + chunk(xrecv, c)[...]
                 + chunk(r_half, c)[...].astype(jnp.bfloat16))
            s = jnp.sum(y * y, axis=-1, keepdims=True, dtype=jnp.float32)
            inv = lax.rsqrt(s * (1.0 / D) + 1e-6)
            out_bf = y * inv.astype(jnp.bfloat16) * gb
            chunk(ysend, c)[...] = out_bf
            r = pltpu.make_async_remote_copy(
                src_ref=chunk(ysend, c), dst_ref=chunk(yrecv, c),
                send_sem=sem_sy.at[c], recv_sem=sem_ry.at[c],
                device_id=y_nbr, device_id_type=pl.DeviceIdType.MESH,
            )
            r.start()
            rdma_y.append(r)
            chunk(p_half, c)[...] = out_bf.astype(jnp.float32)
            co = pltpu.make_async_copy(
                chunk(p_half, c),
                out_ref.at[pl.ds(row0 + c * ROWS, ROWS), :],
                sem_lp.at[c])
            co.start()
            cp_out0.append(co)
            if c >= 2:
                drain_y(c - 2)

        drain_y(NCHUNK - 2)
        drain_y(NCHUNK - 1)

        for c in range(NCHUNK):
            cp_out0[c].wait()
            cp_out1[c].wait()

    return pl.pallas_call(
        body,
        out_shape=jax.ShapeDtypeStruct((M, D), jnp.float32),
        in_specs=[
            pl.BlockSpec(memory_space=pl.ANY),
            pl.BlockSpec(memory_space=pl.ANY),
            pl.BlockSpec(memory_space=pltpu.VMEM),
        ],
        out_specs=pl.BlockSpec(memory_space=pl.ANY),
        scratch_shapes=[
            pltpu.VMEM((HALF, D), jnp.float32),
            pltpu.VMEM((HALF, D), jnp.float32),
            pltpu.VMEM((HALF, D), jnp.bfloat16),
            pltpu.VMEM((HALF, D), jnp.bfloat16),
            pltpu.VMEM((HALF, D), jnp.bfloat16),
            pltpu.VMEM((HALF, D), jnp.bfloat16),
            pltpu.SemaphoreType.DMA((NCHUNK,)),
            pltpu.SemaphoreType.DMA((NCHUNK,)),
            pltpu.SemaphoreType.DMA((NCHUNK,)),
            pltpu.SemaphoreType.DMA((NCHUNK,)),
            pltpu.SemaphoreType.DMA((NCHUNK,)),
            pltpu.SemaphoreType.DMA((NCHUNK,)),
        ],
        compiler_params=pltpu.CompilerParams(
            collective_id=0,
            vmem_limit_bytes=60 * 1024 * 1024,
        ),
    )(partial, resid, gamma)


# device time: 69779 ns/iter; 1.7201x vs baseline; 1.0003x over previous
import os

import jax
import jax.numpy as jnp
from jax import lax
from jax.experimental import pallas as pl
from jax.experimental.pallas import tpu as pltpu

_SKIP_Y = os.environ.get("PROBE_SKIP_Y") == "1"
_SKIP_COMPUTE = os.environ.get("PROBE_SKIP_COMPUTE") == "1"
_SKIP_X = os.environ.get("PROBE_SKIP_X") == "1"
_BARE = os.environ.get("PROBE_BARE") == "1"

M = 2048
D = 2048
HALF = M // 2
NCHUNK = 16
ROWS = HALF // NCHUNK


def kernel(partial, resid, gamma):
    partial = partial.reshape(M, D)
    gamma = gamma.reshape(1, D)

    def body(p_ref, r_ref, g_ref, out_ref,
             p_half, r_half, xsend, xrecv, ysend, yrecv,
             sem_lp, sem_lr, sem_sx, sem_rx, sem_sy, sem_ry):
        my_x = lax.axis_index("x")
        my_y = lax.axis_index("y")
        x_nbr = (1 - my_x, my_y)
        y_nbr = (my_x, 1 - my_y)
        row0 = my_y * HALF
        other_row0 = (1 - my_y) * HALF

        def chunk(buf, c):
            return buf.at[pl.ds(c * ROWS, ROWS), :]

        cp_p = []
        cp_r = []
        for c in range(NCHUNK):
            cp = pltpu.make_async_copy(
                p_ref.at[pl.ds(row0 + c * ROWS, ROWS), :],
                chunk(p_half, c), sem_lp.at[c])
            cp.start()
            cp_p.append(cp)
            cr = pltpu.make_async_copy(
                r_ref.at[pl.ds(row0 + c * ROWS, ROWS), :],
                chunk(r_half, c), sem_lr.at[c])
            cr.start()
            cp_r.append(cr)

        barrier_sem = pltpu.get_barrier_semaphore()
        for nbr in (x_nbr, y_nbr):
            pl.semaphore_signal(
                barrier_sem, inc=1,
                device_id=nbr, device_id_type=pl.DeviceIdType.MESH,
            )
        pl.semaphore_wait(barrier_sem, 2)

        rdma_x = []
        for c in range(NCHUNK):
            cp_p[c].wait()
            chunk(xsend, c)[...] = chunk(p_half, c)[...].astype(jnp.bfloat16)
            if not _SKIP_X:
                r = pltpu.make_async_remote_copy(
                    src_ref=chunk(xsend, c), dst_ref=chunk(xrecv, c),
                    send_sem=sem_sx.at[c], recv_sem=sem_rx.at[c],
                    device_id=x_nbr, device_id_type=pl.DeviceIdType.MESH,
                )
                r.start()
                rdma_x.append(r)

        gb = g_ref[...].astype(jnp.bfloat16)
        rdma_y = []
        cp_out0 = []
        cp_out1 = []

        def drain_y(c):
            rdma_y[c].wait()
            chunk(r_half, c)[...] = chunk(yrecv, c)[...].astype(jnp.float32)
            co = pltpu.make_async_copy(
                chunk(r_half, c),
                out_ref.at[pl.ds(other_row0 + c * ROWS, ROWS), :],
                sem_lr.at[c])
            co.start()
            cp_out1.append(co)

        for c in range(NCHUNK):
            if not _SKIP_X:
                rdma_x[c].wait()
            cp_r[c].wait()
            src = xsend if _SKIP_X else xrecv
            if _SKIP_COMPUTE:
                out_bf = chunk(src, c)[...]
            else:
                y = (chunk(xsend, c)[...] + chunk(src, c)[...]
                     + chunk(r_half, c)[...].astype(jnp.bfloat16))
                s = jnp.sum(y * y, axis=-1, keepdims=True, dtype=jnp.float32)
                inv = lax.rsqrt(s * (1.0 / D) + 1e-6)
                out_bf = y * inv.astype(jnp.bfloat16) * gb
            if not _BARE:
                chunk(ysend, c)[...] = out_bf
            if not _SKIP_Y:
                r = pltpu.make_async_remote_copy(
                    src_ref=chunk(ysend, c), dst_ref=chunk(yrecv, c),
                    send_sem=sem_sy.at[c], recv_sem=sem_ry.at[c],
                    device_id=y_nbr, device_id_type=pl.DeviceIdType.MESH,
                )
                r.start()
                rdma_y.append(r)
            if not _BARE:
                chunk(p_half, c)[...] = out_bf.astype(jnp.float32)
                co = pltpu.make_async_copy(
                    chunk(p_half, c),
                    out_ref.at[pl.ds(row0 + c * ROWS, ROWS), :],
                    sem_lp.at[c])
                co.start()
                cp_out0.append(co)
            if not _SKIP_Y and c >= 2:
                drain_y(c - 2)

        if not _SKIP_Y:
            drain_y(NCHUNK - 2)
            drain_y(NCHUNK - 1)

        for c in range(NCHUNK):
            if not _BARE:
                cp_out0[c].wait()
            if not _SKIP_Y:
                cp_out1[c].wait()

    return pl.pallas_call(
        body,
        out_shape=jax.ShapeDtypeStruct((M, D), jnp.float32),
        in_specs=[
            pl.BlockSpec(memory_space=pl.ANY),
            pl.BlockSpec(memory_space=pl.ANY),
            pl.BlockSpec(memory_space=pltpu.VMEM),
        ],
        out_specs=pl.BlockSpec(memory_space=pl.ANY),
        scratch_shapes=[
            pltpu.VMEM((HALF, D), jnp.float32),
            pltpu.VMEM((HALF, D), jnp.float32),
            pltpu.VMEM((HALF, D), jnp.bfloat16),
            pltpu.VMEM((HALF, D), jnp.bfloat16),
            pltpu.VMEM((HALF, D), jnp.bfloat16),
            pltpu.VMEM((HALF, D), jnp.bfloat16),
            pltpu.SemaphoreType.DMA((NCHUNK,)),
            pltpu.SemaphoreType.DMA((NCHUNK,)),
            pltpu.SemaphoreType.DMA((NCHUNK,)),
            pltpu.SemaphoreType.DMA((NCHUNK,)),
            pltpu.SemaphoreType.DMA((NCHUNK,)),
            pltpu.SemaphoreType.DMA((NCHUNK,)),
        ],
        compiler_params=pltpu.CompilerParams(
            collective_id=0,
            vmem_limit_bytes=60 * 1024 * 1024,
        ),
    )(partial, resid, gamma)
